# baseline (speedup 1.0000x reference)
import functools

import jax
import jax.numpy as jnp
from jax.experimental import pallas as pl

B, C, D, H, W = 4, 128, 2, 200, 176
HW = H * W
PC = HW // 128
NSLOT = B * D * HW


def _tc_body(a0, a1, s0, s1, out_ref):
    for d, (a, s) in enumerate(((a0, s0), (a1, s1))):
        t = a[...].T
        valid = (s[...] >= 0).reshape(1, 128)
        out_ref[0, :, d, :] = jnp.where(valid, t, 0.0)


@jax.jit
def _tc_transpose(a, sv3):
    return pl.pallas_call(
        _tc_body,
        grid=(B, PC),
        in_specs=[
            pl.BlockSpec((128, 128), lambda b, p: (b * 2 * PC + p, 0)),
            pl.BlockSpec((128, 128), lambda b, p: (b * 2 * PC + PC + p, 0)),
            pl.BlockSpec((1, 1, 128), lambda b, p: (b * 2 * PC + p, 0, 0)),
            pl.BlockSpec((1, 1, 128), lambda b, p: (b * 2 * PC + PC + p, 0, 0)),
        ],
        out_specs=pl.BlockSpec((1, 128, 2, 128), lambda b, p: (b, 0, 0, p)),
        out_shape=jax.ShapeDtypeStruct((B, 128, 2, HW), jnp.float32),
    )(a, a, sv3, sv3)


def kernel(voxel_features, voxel_lin_idx):
    lin = voxel_lin_idx.astype(jnp.int32)
    dense = jnp.zeros((NSLOT, C), dtype=voxel_features.dtype)
    dense = dense.at[lin].set(voxel_features)
    sv3 = jnp.zeros((B * D * PC, 1, 128), dtype=jnp.int32)
    out = _tc_transpose(dense, sv3)
    return out.reshape(B, C * D, H, W)
